# Initial kernel scaffold; baseline (speedup 1.0000x reference)
import jax
import jax.numpy as jnp
from jax.experimental import pallas as pl


def kernel(input_ids, token_type_ids, word_table, pos_table, type_table, gamma, beta):
    raise NotImplementedError("write your pallas kernel here")


# speedup vs baseline: 2.0777x; 2.0777x over previous
import functools

import jax
import jax.numpy as jnp
from jax import lax
from jax.experimental import pallas as pl
from jax.experimental.pallas import tpu as pltpu
from jax.experimental.pallas import tpu_sc as plsc

H = 768
EPS = 1e-12

NC = 2
NS = 16
NW = NC * NS

TC_BLK = 256


def _sc_gather(word_table, flat_ids, n_tok):
    b_per_w = n_tok // NW
    chunk = 64
    nchunk = b_per_w // chunk
    mesh = plsc.VectorSubcoreMesh(core_axis_name="c", subcore_axis_name="s")

    @functools.partial(
        pl.kernel,
        mesh=mesh,
        out_type=jax.ShapeDtypeStruct((n_tok, H), jnp.float32),
        scratch_types=[
            pltpu.VMEM((b_per_w,), jnp.int32),
            pltpu.VMEM((2, chunk, H), jnp.float32),
            pltpu.SemaphoreType.DMA,
            pltpu.SemaphoreType.DMA,
        ],
    )
    def gather_kernel(table_hbm, idx_hbm, out_hbm, idx_v, rows_v, sem_in, sem_out):
        wid = lax.axis_index("s") * NC + lax.axis_index("c")
        base = wid * b_per_w
        pltpu.sync_copy(idx_hbm.at[pl.ds(base, b_per_w)], idx_v)

        def g(c):
            return pltpu.make_async_copy(
                table_hbm.at[idx_v.at[pl.ds(c * chunk, chunk)]],
                rows_v.at[c % 2],
                sem_in,
            )

        def p(c):
            return pltpu.make_async_copy(
                rows_v.at[c % 2],
                out_hbm.at[pl.ds(base + c * chunk, chunk)],
                sem_out,
            )

        g(0).start()
        for c in range(nchunk):
            g(c).wait()
            if c + 1 < nchunk:
                if c >= 1:
                    p(c - 1).wait()
                g(c + 1).start()
            p(c).start()
        p(nchunk - 1).wait()
        if nchunk >= 2:
            p(nchunk - 2).wait()

    return gather_kernel(word_table, flat_ids)


def _tc_body(words_ref, pos_ref, ttf_ref, type_ref, gamma_ref, beta_ref, out_ref):
    t0 = type_ref[0:1, :]
    t1 = type_ref[1:2, :]
    x = words_ref[...] + pos_ref[...] + t0 + ttf_ref[...] * (t1 - t0)
    u = jnp.mean(x, axis=1, keepdims=True)
    xc = x - u
    s = jnp.mean(xc * xc, axis=1, keepdims=True)
    y = xc * lax.rsqrt(s + EPS)
    out_ref[...] = gamma_ref[...] * y + beta_ref[...]


def kernel(input_ids, token_type_ids, word_table, pos_table, type_table, gamma, beta):
    b, s = input_ids.shape
    n_tok = b * s
    flat_ids = input_ids.reshape(n_tok).astype(jnp.int32)

    words = _sc_gather(word_table, flat_ids, n_tok)

    ttf = token_type_ids.reshape(n_tok, 1).astype(jnp.float32)
    nblk = n_tok // TC_BLK
    pos_blocks = s // TC_BLK

    out = pl.pallas_call(
        _tc_body,
        grid=(nblk,),
        in_specs=[
            pl.BlockSpec((TC_BLK, H), lambda i: (i, 0)),
            pl.BlockSpec((TC_BLK, H), lambda i: (i % pos_blocks, 0)),
            pl.BlockSpec((TC_BLK, 1), lambda i: (i, 0)),
            pl.BlockSpec((2, H), lambda i: (0, 0)),
            pl.BlockSpec((1, H), lambda i: (0, 0)),
            pl.BlockSpec((1, H), lambda i: (0, 0)),
        ],
        out_specs=pl.BlockSpec((TC_BLK, H), lambda i: (i, 0)),
        out_shape=jax.ShapeDtypeStruct((n_tok, H), jnp.float32),
    )(
        words,
        pos_table,
        ttf,
        type_table,
        gamma.reshape(1, H),
        beta.reshape(1, H),
    )

    return (out.reshape(b, s, H), words.reshape(b, s, H))


# speedup vs baseline: 2.1001x; 1.0107x over previous
import functools

import jax
import jax.numpy as jnp
from jax import lax
from jax.experimental import pallas as pl
from jax.experimental.pallas import tpu as pltpu
from jax.experimental.pallas import tpu_sc as plsc

H = 768
EPS = 1e-12

NC = 2
NS = 16
NW = NC * NS

TC_BLK = 256


def _sc_gather(word_table, flat_ids, n_tok):
    b_per_w = n_tok // NW
    chunk = 64
    nchunk = b_per_w // chunk
    mesh = plsc.VectorSubcoreMesh(core_axis_name="c", subcore_axis_name="s")

    @functools.partial(
        pl.kernel,
        mesh=mesh,
        out_type=jax.ShapeDtypeStruct((n_tok, H), jnp.float32),
        scratch_types=[
            pltpu.VMEM((b_per_w,), jnp.int32),
            pltpu.VMEM((2, chunk, H), jnp.float32),
            pltpu.SemaphoreType.DMA,
            pltpu.SemaphoreType.DMA,
        ],
    )
    def gather_kernel(table_hbm, idx_hbm, out_hbm, idx_v, rows_v, sem_in, sem_out):
        wid = lax.axis_index("s") * NC + lax.axis_index("c")
        base = wid * b_per_w
        pltpu.sync_copy(idx_hbm.at[pl.ds(base, b_per_w)], idx_v)

        def g(c):
            return pltpu.make_async_copy(
                table_hbm.at[idx_v.at[pl.ds(c * chunk, chunk)]],
                rows_v.at[c % 2],
                sem_in,
            )

        def p(c):
            return pltpu.make_async_copy(
                rows_v.at[c % 2],
                out_hbm.at[pl.ds(base + c * chunk, chunk)],
                sem_out,
            )

        g(0).start()
        for c in range(nchunk):
            g(c).wait()
            if c + 1 < nchunk:
                if c >= 1:
                    p(c - 1).wait()
                g(c + 1).start()
            p(c).start()
        p(nchunk - 1).wait()
        if nchunk >= 2:
            p(nchunk - 2).wait()

    return gather_kernel(word_table, flat_ids)


def _tc_body(words_ref, pos_ref, ttf_ref, type_ref, gamma_ref, beta_ref, out_ref):
    t0 = type_ref[0:1, :]
    t1 = type_ref[1:2, :]
    x = words_ref[...] + pos_ref[...] + t0 + ttf_ref[...] * (t1 - t0)
    u = jnp.mean(x, axis=1, keepdims=True)
    xc = x - u
    s = jnp.mean(xc * xc, axis=1, keepdims=True)
    y = xc * lax.rsqrt(s + EPS)
    out_ref[...] = gamma_ref[...] * y + beta_ref[...]


def kernel(input_ids, token_type_ids, word_table, pos_table, type_table, gamma, beta):
    b, s = input_ids.shape
    n_tok = b * s
    flat_ids = input_ids.reshape(n_tok).astype(jnp.int32)

    words = _sc_gather(word_table, flat_ids, n_tok)

    ttf = token_type_ids.reshape(n_tok, 1).astype(jnp.float32)
    pos_blocks = s // TC_BLK

    out = pl.pallas_call(
        _tc_body,
        grid=(pos_blocks, b),
        in_specs=[
            pl.BlockSpec((TC_BLK, H), lambda p, j: (j * pos_blocks + p, 0)),
            pl.BlockSpec((TC_BLK, H), lambda p, j: (p, 0)),
            pl.BlockSpec((TC_BLK, 1), lambda p, j: (j * pos_blocks + p, 0)),
            pl.BlockSpec((2, H), lambda p, j: (0, 0)),
            pl.BlockSpec((1, H), lambda p, j: (0, 0)),
            pl.BlockSpec((1, H), lambda p, j: (0, 0)),
        ],
        out_specs=pl.BlockSpec((TC_BLK, H), lambda p, j: (j * pos_blocks + p, 0)),
        out_shape=jax.ShapeDtypeStruct((n_tok, H), jnp.float32),
        compiler_params=pltpu.CompilerParams(
            dimension_semantics=("parallel", "arbitrary"),
        ),
    )(
        words,
        pos_table,
        ttf,
        type_table,
        gamma.reshape(1, H),
        beta.reshape(1, H),
    )

    return (out.reshape(b, s, H), words.reshape(b, s, H))


# speedup vs baseline: 2.4340x; 1.1590x over previous
import functools

import jax
import jax.numpy as jnp
from jax import lax
from jax.experimental import pallas as pl
from jax.experimental.pallas import tpu as pltpu
from jax.experimental.pallas import tpu_sc as plsc

H = 768
EPS = 1e-12

NC = 2
NS = 16
NW = NC * NS

TC_BLK = 512


def _sc_gather(word_table, flat_ids, n_tok):
    b_per_w = n_tok // NW
    chunk = 64
    nchunk = b_per_w // chunk
    mesh = plsc.VectorSubcoreMesh(core_axis_name="c", subcore_axis_name="s")

    @functools.partial(
        pl.kernel,
        mesh=mesh,
        out_type=jax.ShapeDtypeStruct((n_tok, H), jnp.float32),
        scratch_types=[
            pltpu.VMEM((b_per_w,), jnp.int32),
            pltpu.VMEM((2, chunk, H), jnp.float32),
            pltpu.SemaphoreType.DMA,
            pltpu.SemaphoreType.DMA,
        ],
    )
    def gather_kernel(table_hbm, idx_hbm, out_hbm, idx_v, rows_v, sem_in, sem_out):
        wid = lax.axis_index("s") * NC + lax.axis_index("c")
        base = wid * b_per_w
        pltpu.sync_copy(idx_hbm.at[pl.ds(base, b_per_w)], idx_v)

        def g(c):
            return pltpu.make_async_copy(
                table_hbm.at[idx_v.at[pl.ds(c * chunk, chunk)]],
                rows_v.at[c % 2],
                sem_in,
            )

        def p(c):
            return pltpu.make_async_copy(
                rows_v.at[c % 2],
                out_hbm.at[pl.ds(base + c * chunk, chunk)],
                sem_out,
            )

        g(0).start()
        for c in range(nchunk):
            g(c).wait()
            if c + 1 < nchunk:
                if c >= 1:
                    p(c - 1).wait()
                g(c + 1).start()
            p(c).start()
        p(nchunk - 1).wait()
        if nchunk >= 2:
            p(nchunk - 2).wait()

    return gather_kernel(word_table, flat_ids)


def _tc_body(words_ref, pos_ref, ttf_ref, type_ref, gamma_ref, beta_ref, out_ref):
    t0 = type_ref[0:1, :]
    t1 = type_ref[1:2, :]
    x = words_ref[...] + pos_ref[...] + t0 + ttf_ref[...] * (t1 - t0)
    u = jnp.mean(x, axis=1, keepdims=True)
    xc = x - u
    s = jnp.mean(xc * xc, axis=1, keepdims=True)
    y = xc * lax.rsqrt(s + EPS)
    out_ref[...] = gamma_ref[...] * y + beta_ref[...]


def kernel(input_ids, token_type_ids, word_table, pos_table, type_table, gamma, beta):
    b, s = input_ids.shape
    n_tok = b * s
    flat_ids = input_ids.reshape(n_tok).astype(jnp.int32)

    words = _sc_gather(word_table, flat_ids, n_tok)

    ttf = token_type_ids.reshape(n_tok, 1).astype(jnp.float32)
    pos_blocks = s // TC_BLK

    out = pl.pallas_call(
        _tc_body,
        grid=(pos_blocks, b),
        in_specs=[
            pl.BlockSpec((TC_BLK, H), lambda p, j: (j * pos_blocks + p, 0)),
            pl.BlockSpec((TC_BLK, H), lambda p, j: (p, 0)),
            pl.BlockSpec((TC_BLK, 1), lambda p, j: (j * pos_blocks + p, 0)),
            pl.BlockSpec((2, H), lambda p, j: (0, 0)),
            pl.BlockSpec((1, H), lambda p, j: (0, 0)),
            pl.BlockSpec((1, H), lambda p, j: (0, 0)),
        ],
        out_specs=pl.BlockSpec((TC_BLK, H), lambda p, j: (j * pos_blocks + p, 0)),
        out_shape=jax.ShapeDtypeStruct((n_tok, H), jnp.float32),
        compiler_params=pltpu.CompilerParams(
            dimension_semantics=("parallel", "arbitrary"),
        ),
    )(
        words,
        pos_table,
        ttf,
        type_table,
        gamma.reshape(1, H),
        beta.reshape(1, H),
    )

    return (out.reshape(b, s, H), words.reshape(b, s, H))


# speedup vs baseline: 2.5576x; 1.0508x over previous
import functools

import jax
import jax.numpy as jnp
from jax import lax
from jax.experimental import pallas as pl
from jax.experimental.pallas import tpu as pltpu
from jax.experimental.pallas import tpu_sc as plsc

H = 768
EPS = 1e-12

NC = 2
NS = 16
NW = NC * NS

TC_BLK = 1024


def _sc_gather(word_table, flat_ids, n_tok):
    b_per_w = n_tok // NW
    chunk = 64
    nchunk = b_per_w // chunk
    mesh = plsc.VectorSubcoreMesh(core_axis_name="c", subcore_axis_name="s")

    @functools.partial(
        pl.kernel,
        mesh=mesh,
        out_type=jax.ShapeDtypeStruct((n_tok, H), jnp.float32),
        scratch_types=[
            pltpu.VMEM((b_per_w,), jnp.int32),
            pltpu.VMEM((2, chunk, H), jnp.float32),
            pltpu.SemaphoreType.DMA,
            pltpu.SemaphoreType.DMA,
        ],
    )
    def gather_kernel(table_hbm, idx_hbm, out_hbm, idx_v, rows_v, sem_in, sem_out):
        wid = lax.axis_index("s") * NC + lax.axis_index("c")
        base = wid * b_per_w
        pltpu.sync_copy(idx_hbm.at[pl.ds(base, b_per_w)], idx_v)

        def g(c):
            return pltpu.make_async_copy(
                table_hbm.at[idx_v.at[pl.ds(c * chunk, chunk)]],
                rows_v.at[c % 2],
                sem_in,
            )

        def p(c):
            return pltpu.make_async_copy(
                rows_v.at[c % 2],
                out_hbm.at[pl.ds(base + c * chunk, chunk)],
                sem_out,
            )

        g(0).start()
        for c in range(nchunk):
            g(c).wait()
            if c + 1 < nchunk:
                if c >= 1:
                    p(c - 1).wait()
                g(c + 1).start()
            p(c).start()
        p(nchunk - 1).wait()
        if nchunk >= 2:
            p(nchunk - 2).wait()

    return gather_kernel(word_table, flat_ids)


def _tc_body(words_ref, pos_ref, ttf_ref, type_ref, gamma_ref, beta_ref, out_ref):
    t0 = type_ref[0:1, :]
    t1 = type_ref[1:2, :]
    x = words_ref[...] + pos_ref[...] + t0 + ttf_ref[...] * (t1 - t0)
    u = jnp.mean(x, axis=1, keepdims=True)
    xc = x - u
    s = jnp.mean(xc * xc, axis=1, keepdims=True)
    y = xc * lax.rsqrt(s + EPS)
    out_ref[...] = gamma_ref[...] * y + beta_ref[...]


def kernel(input_ids, token_type_ids, word_table, pos_table, type_table, gamma, beta):
    b, s = input_ids.shape
    n_tok = b * s
    flat_ids = input_ids.reshape(n_tok).astype(jnp.int32)

    words = _sc_gather(word_table, flat_ids, n_tok)

    ttf = token_type_ids.reshape(n_tok, 1).astype(jnp.float32)
    pos_blocks = s // TC_BLK

    out = pl.pallas_call(
        _tc_body,
        grid=(pos_blocks, b),
        in_specs=[
            pl.BlockSpec((TC_BLK, H), lambda p, j: (j * pos_blocks + p, 0)),
            pl.BlockSpec((TC_BLK, H), lambda p, j: (p, 0)),
            pl.BlockSpec((TC_BLK, 1), lambda p, j: (j * pos_blocks + p, 0)),
            pl.BlockSpec((2, H), lambda p, j: (0, 0)),
            pl.BlockSpec((1, H), lambda p, j: (0, 0)),
            pl.BlockSpec((1, H), lambda p, j: (0, 0)),
        ],
        out_specs=pl.BlockSpec((TC_BLK, H), lambda p, j: (j * pos_blocks + p, 0)),
        out_shape=jax.ShapeDtypeStruct((n_tok, H), jnp.float32),
        compiler_params=pltpu.CompilerParams(
            dimension_semantics=("parallel", "arbitrary"),
        ),
    )(
        words,
        pos_table,
        ttf,
        type_table,
        gamma.reshape(1, H),
        beta.reshape(1, H),
    )

    return (out.reshape(b, s, H), words.reshape(b, s, H))


# speedup vs baseline: 2.5811x; 1.0092x over previous
import functools

import jax
import jax.numpy as jnp
from jax import lax
from jax.experimental import pallas as pl
from jax.experimental.pallas import tpu as pltpu
from jax.experimental.pallas import tpu_sc as plsc

H = 768
EPS = 1e-12

NC = 2
NS = 16
NW = NC * NS

TC_BLK = 2048


def _sc_gather(word_table, flat_ids, n_tok):
    b_per_w = n_tok // NW
    chunk = 64
    nchunk = b_per_w // chunk
    mesh = plsc.VectorSubcoreMesh(core_axis_name="c", subcore_axis_name="s")

    @functools.partial(
        pl.kernel,
        mesh=mesh,
        out_type=jax.ShapeDtypeStruct((n_tok, H), jnp.float32),
        scratch_types=[
            pltpu.VMEM((b_per_w,), jnp.int32),
            pltpu.VMEM((2, chunk, H), jnp.float32),
            pltpu.SemaphoreType.DMA,
            pltpu.SemaphoreType.DMA,
        ],
    )
    def gather_kernel(table_hbm, idx_hbm, out_hbm, idx_v, rows_v, sem_in, sem_out):
        wid = lax.axis_index("s") * NC + lax.axis_index("c")
        base = wid * b_per_w
        pltpu.sync_copy(idx_hbm.at[pl.ds(base, b_per_w)], idx_v)

        def g(c):
            return pltpu.make_async_copy(
                table_hbm.at[idx_v.at[pl.ds(c * chunk, chunk)]],
                rows_v.at[c % 2],
                sem_in,
            )

        def p(c):
            return pltpu.make_async_copy(
                rows_v.at[c % 2],
                out_hbm.at[pl.ds(base + c * chunk, chunk)],
                sem_out,
            )

        g(0).start()
        for c in range(nchunk):
            g(c).wait()
            if c + 1 < nchunk:
                if c >= 1:
                    p(c - 1).wait()
                g(c + 1).start()
            p(c).start()
        p(nchunk - 1).wait()
        if nchunk >= 2:
            p(nchunk - 2).wait()

    return gather_kernel(word_table, flat_ids)


def _tc_body(words_ref, pos_ref, ttf_ref, type_ref, gamma_ref, beta_ref, out_ref):
    t0 = type_ref[0:1, :]
    t1 = type_ref[1:2, :]
    x = words_ref[...] + pos_ref[...] + t0 + ttf_ref[...] * (t1 - t0)
    u = jnp.mean(x, axis=1, keepdims=True)
    xc = x - u
    s = jnp.mean(xc * xc, axis=1, keepdims=True)
    y = xc * lax.rsqrt(s + EPS)
    out_ref[...] = gamma_ref[...] * y + beta_ref[...]


def kernel(input_ids, token_type_ids, word_table, pos_table, type_table, gamma, beta):
    b, s = input_ids.shape
    n_tok = b * s
    flat_ids = input_ids.reshape(n_tok).astype(jnp.int32)

    words = _sc_gather(word_table, flat_ids, n_tok)

    ttf = token_type_ids.reshape(n_tok, 1).astype(jnp.float32)
    pos_blocks = s // TC_BLK

    out = pl.pallas_call(
        _tc_body,
        grid=(pos_blocks, b),
        in_specs=[
            pl.BlockSpec((TC_BLK, H), lambda p, j: (j * pos_blocks + p, 0)),
            pl.BlockSpec((TC_BLK, H), lambda p, j: (p, 0)),
            pl.BlockSpec((TC_BLK, 1), lambda p, j: (j * pos_blocks + p, 0)),
            pl.BlockSpec((2, H), lambda p, j: (0, 0)),
            pl.BlockSpec((1, H), lambda p, j: (0, 0)),
            pl.BlockSpec((1, H), lambda p, j: (0, 0)),
        ],
        out_specs=pl.BlockSpec((TC_BLK, H), lambda p, j: (j * pos_blocks + p, 0)),
        out_shape=jax.ShapeDtypeStruct((n_tok, H), jnp.float32),
        compiler_params=pltpu.CompilerParams(
            dimension_semantics=("parallel", "arbitrary"),
        ),
    )(
        words,
        pos_table,
        ttf,
        type_table,
        gamma.reshape(1, H),
        beta.reshape(1, H),
    )

    return (out.reshape(b, s, H), words.reshape(b, s, H))
